# Initial kernel scaffold; baseline (speedup 1.0000x reference)
import jax
import jax.numpy as jnp
from jax.experimental import pallas as pl


def kernel(x, item_table, pos_table):
    raise NotImplementedError("write your pallas kernel here")


# speedup vs baseline: 2.0578x; 2.0578x over previous
import functools

import jax
import jax.numpy as jnp
from jax import lax
from jax.experimental import pallas as pl
from jax.experimental.pallas import tpu as pltpu
from jax.experimental.pallas import tpu_sc as plsc

NUM_ITEM = 1000000
HIDDEN = 64
SEQ = 200
BATCH = 4096
PAD = 3

NC, NS, L = 2, 16, 16
NW = NC * NS
N = BATCH * SEQ
PER_W = N // NW
C = 800
G = PER_W // C
PIECES = [(o, min(128, C - o)) for o in range(0, C, 128)]


def _body(x_hbm, tbl_hbm, aux_hbm, posm_hbm, out_hbm, mask_hbm,
          idx_v, auxi_v, mask_v, posm_v, dest_v,
          sem_g, sem_a):
    wid = lax.axis_index("s") * NC + lax.axis_index("c")
    w0 = wid * PER_W

    pltpu.sync_copy(posm_hbm, posm_v)

    @pl.loop(0, G)
    def _chunk(g):
        base = w0 + g * C
        pltpu.sync_copy(x_hbm.at[pl.ds(base, C)], idx_v)

        for j in range(C // L):
            sl = pl.ds(j * L, L)
            iv = idx_v[sl]
            pv = posm_v[sl]
            pad = iv == PAD
            auxi_v[sl] = jnp.where(pad, SEQ, pv)
            mask_v[sl] = jnp.where(pad, 1, 0)

        descs = [
            pltpu.async_copy(aux_hbm.at[auxi_v.at[pl.ds(o, s)]],
                             dest_v.at[pl.ds(o, s)], sem_g)
            for o, s in PIECES
        ]
        for d in descs:
            d.wait()

        descs = [
            pltpu.async_copy(tbl_hbm.at[idx_v.at[pl.ds(o, s)]],
                             dest_v.at[pl.ds(o, s)], sem_a, add=True)
            for o, s in PIECES
        ]
        for d in descs:
            d.wait()

        pltpu.sync_copy(dest_v, out_hbm.at[pl.ds(base, C)])
        pltpu.sync_copy(mask_v, mask_hbm.at[pl.ds(base, C)])


@jax.jit
def _sc_embed(xf, item_table, aux, posm):
    return pl.kernel(
        _body,
        out_type=[
            jax.ShapeDtypeStruct((N, HIDDEN), jnp.float32),
            jax.ShapeDtypeStruct((N,), jnp.int32),
        ],
        mesh=plsc.VectorSubcoreMesh(
            core_axis_name="c", subcore_axis_name="s",
            num_cores=NC, num_subcores=NS),
        compiler_params=pltpu.CompilerParams(use_tc_tiling_on_sc=False),
        scratch_types=[
            pltpu.VMEM((C,), jnp.int32),
            pltpu.VMEM((C,), jnp.int32),
            pltpu.VMEM((C,), jnp.int32),
            pltpu.VMEM((C,), jnp.int32),
            pltpu.VMEM((C, HIDDEN), jnp.float32),
            pltpu.SemaphoreType.DMA,
            pltpu.SemaphoreType.DMA,
        ],
    )(xf, item_table, aux, posm)


def kernel(x, item_table, pos_table):
    xf = x.reshape(N)
    aux = jnp.concatenate([pos_table, -item_table[PAD:PAD + 1]], axis=0)
    posm = jnp.tile(jnp.arange(SEQ, dtype=jnp.int32), C // SEQ)
    emb, mask = _sc_embed(xf, item_table, aux, posm)
    return (emb.reshape(BATCH, SEQ, HIDDEN),
            mask.reshape(BATCH, SEQ).astype(bool))
